# Initial kernel scaffold; baseline (speedup 1.0000x reference)
import functools

import jax
import jax.numpy as jnp
from jax import lax
from jax.experimental import pallas as pl
from jax.experimental.pallas import tpu as pltpu
from jax.experimental.pallas import tpu_sc as plsc

_N_NODES = 100000
_N_EDGES = 6400000
_NC = 2
_NS = 16
_NW = _NC * _NS
_B = 2048
_NBLK = _N_EDGES // _B
_STEPS = _B // 16
_N_PAD = 100352
_ZSL = _N_PAD // _NS
_CSL = _N_PAD // _NW

_KEHALF = 7.199822675975274
_C1 = 1.0 / 144.0
_C2 = 1.0 / 6.0
_CUTOFF = 12.0

_mesh = plsc.VectorSubcoreMesh(
    core_axis_name="c", subcore_axis_name="s", num_cores=_NC, num_subcores=_NS
)


def _energy(d, qi, qj):
    t = d * d + 1.0
    ib = plsc.bitcast(t, jnp.int32)
    ib = jnp.int32(0x5F3759DF) - lax.shift_right_arithmetic(ib, 1)
    r = plsc.bitcast(ib, jnp.float32)
    th = 0.5 * t
    r = r * (1.5 - th * r * r)
    r = r * (1.5 - th * r * r)
    r = r * (1.5 - th * r * r)
    ds_ = t * r
    e_shd = r + (ds_ * jnp.float32(_C1) - jnp.float32(_C2))
    e_ord = 1.0 / d + (d * jnp.float32(_C1) - jnp.float32(_C2))
    x = jnp.minimum(d * 0.5, 1.0)
    x2 = x * x
    x3 = x2 * x
    w_on = x3 * ((6.0 * x2 - 15.0 * x) + 10.0)
    mix = e_shd + w_on * (e_ord - e_shd)
    e = (jnp.float32(_KEHALF) * qi) * qj * mix
    return jnp.where(d <= jnp.float32(_CUTOFF), e, jnp.float32(0.0))


@functools.partial(
    pl.kernel,
    out_type=jax.ShapeDtypeStruct((_NC, _N_PAD), jnp.float32),
    mesh=_mesh,
    scratch_types=[
        pltpu.VMEM((_N_NODES,), jnp.float32),
        pltpu.VMEM((_B,), jnp.float32),
        pltpu.VMEM((_B,), jnp.int32),
        pltpu.VMEM((_B,), jnp.int32),
        pltpu.VMEM((_B,), jnp.float32),
        pltpu.VMEM_SHARED((_N_PAD,), jnp.float32),
    ],
)
def _sc_main(ch_hbm, d_hbm, ii_hbm, ij_hbm, out_hbm,
             table_v, d_v, ii_v, ij_v, e_v, accum_sh):
    cid = lax.axis_index("c")
    sid = lax.axis_index("s")
    wid = sid * _NC + cid

    pltpu.sync_copy(ch_hbm, table_v)

    def zstep(k, _):
        e_v[pl.ds(k * 16, 16)] = jnp.zeros((16,), jnp.float32)
        return 0
    lax.fori_loop(0, _STEPS, zstep, 0)
    zbase = sid * _ZSL
    pltpu.sync_copy(e_v, accum_sh.at[pl.ds(zbase, _B)])
    pltpu.sync_copy(e_v, accum_sh.at[pl.ds(zbase + _B, _B)])
    pltpu.sync_copy(e_v, accum_sh.at[pl.ds(zbase + 2 * _B, _B)])
    pltpu.sync_copy(e_v.at[pl.ds(0, _ZSL - 3 * _B)],
                    accum_sh.at[pl.ds(zbase + 3 * _B, _ZSL - 3 * _B)])
    plsc.subcore_barrier()

    nblk_mine = (_NBLK - wid + _NW - 1) // _NW

    def blk_body(tt, _):
        off = (wid + tt * _NW) * _B
        pltpu.sync_copy(d_hbm.at[pl.ds(off, _B)], d_v)
        pltpu.sync_copy(ii_hbm.at[pl.ds(off, _B)], ii_v)
        pltpu.sync_copy(ij_hbm.at[pl.ds(off, _B)], ij_v)

        def step(k, _):
            s = pl.ds(k * 16, 16)
            ii = ii_v[s]
            ij = ij_v[s]
            d = d_v[s]
            qi = plsc.load_gather(table_v, [ii])
            qj = plsc.load_gather(table_v, [ij])
            e_v[s] = _energy(d, qi, qj)
            return 0
        lax.fori_loop(0, _STEPS, step, 0)
        pltpu.sync_copy(e_v, accum_sh.at[ii_v], add=True)
        return 0
    lax.fori_loop(0, nblk_mine, blk_body, 0)

    plsc.subcore_barrier()

    def wb(base, n):
        pltpu.sync_copy(accum_sh.at[pl.ds(base, n)], e_v.at[pl.ds(0, n)])
        pltpu.sync_copy(e_v.at[pl.ds(0, n)], out_hbm.at[cid, pl.ds(base, n)])
    wb(zbase, _B)
    wb(zbase + _B, _B)
    wb(zbase + 2 * _B, _B)
    wb(zbase + 3 * _B, _ZSL - 3 * _B)


@functools.partial(
    pl.kernel,
    out_type=jax.ShapeDtypeStruct((_N_PAD,), jnp.float32),
    mesh=_mesh,
    scratch_types=[
        pltpu.VMEM((_CSL,), jnp.float32),
        pltpu.VMEM((_CSL,), jnp.float32),
    ],
)
def _sc_combine(p_hbm, out_hbm, a_v, b_v):
    cid = lax.axis_index("c")
    sid = lax.axis_index("s")
    wid = sid * _NC + cid
    base = wid * _CSL
    pltpu.sync_copy(p_hbm.at[0, pl.ds(base, _CSL)], a_v)
    pltpu.sync_copy(p_hbm.at[1, pl.ds(base, _CSL)], b_v)

    def step(k, _):
        s = pl.ds(k * 16, 16)
        a_v[s] = a_v[s] + b_v[s]
        return 0
    lax.fori_loop(0, _CSL // 16, step, 0)
    pltpu.sync_copy(a_v, out_hbm.at[pl.ds(base, _CSL)])


def kernel(atomic_charges, distances, idx_i, idx_j):
    parts = _sc_main(atomic_charges, distances, idx_i, idx_j)
    summed = _sc_combine(parts)
    return summed[:_N_NODES]


# speedup vs baseline: 328.4406x; 328.4406x over previous
import functools

import jax
import jax.numpy as jnp
from jax import lax
from jax.experimental import pallas as pl
from jax.experimental.pallas import tpu as pltpu
from jax.experimental.pallas import tpu_sc as plsc

_N_NODES = 100000
_N_EDGES = 6400000
_NC = 2
_NS = 16
_NW = _NC * _NS
_B = 2048
_NBLK = _N_EDGES // _B
_STEPS = _B // 16
_N_PAD = 100352
_ZSL = _N_PAD // _NS
_CSL = _N_PAD // _NW

_KEHALF = 7.199822675975274
_C1 = 1.0 / 144.0
_C2 = 1.0 / 6.0
_CUTOFF = 12.0

_mesh = plsc.VectorSubcoreMesh(
    core_axis_name="c", subcore_axis_name="s", num_cores=_NC, num_subcores=_NS
)


def _energy(d, qi, qj):
    t = d * d + 1.0
    ib = plsc.bitcast(t, jnp.int32)
    ib = jnp.int32(0x5F3759DF) - lax.shift_right_arithmetic(ib, 1)
    r = plsc.bitcast(ib, jnp.float32)
    th = 0.5 * t
    r = r * (1.5 - th * r * r)
    r = r * (1.5 - th * r * r)
    r = r * (1.5 - th * r * r)
    ds_ = t * r
    e_shd = r + (ds_ * jnp.float32(_C1) - jnp.float32(_C2))
    e_ord = 1.0 / d + (d * jnp.float32(_C1) - jnp.float32(_C2))
    x = jnp.minimum(d * 0.5, 1.0)
    x2 = x * x
    x3 = x2 * x
    w_on = x3 * ((6.0 * x2 - 15.0 * x) + 10.0)
    mix = e_shd + w_on * (e_ord - e_shd)
    e = (jnp.float32(_KEHALF) * qi) * qj * mix
    return jnp.where(d <= jnp.float32(_CUTOFF), e, jnp.float32(0.0))


@functools.partial(
    pl.kernel,
    out_type=(jax.ShapeDtypeStruct((_N_PAD,), jnp.float32),
              jax.ShapeDtypeStruct((_N_PAD,), jnp.float32)),
    mesh=_mesh,
    compiler_params=pltpu.CompilerParams(needs_layout_passes=False),
    scratch_types=[
        pltpu.VMEM((_N_NODES,), jnp.float32),
        pltpu.VMEM((_B,), jnp.float32),
        pltpu.VMEM((_B,), jnp.int32),
        pltpu.VMEM((_B,), jnp.int32),
        pltpu.VMEM((_B,), jnp.float32),
        pltpu.VMEM_SHARED((_N_PAD,), jnp.float32),
    ],
)
def _sc_main(ch_hbm, d_hbm, ii_hbm, ij_hbm, out0_hbm, out1_hbm,
             table_v, d_v, ii_v, ij_v, e_v, accum_sh):
    cid = lax.axis_index("c")
    sid = lax.axis_index("s")
    wid = sid * _NC + cid

    pltpu.sync_copy(ch_hbm, table_v)

    def zstep(k, _):
        e_v[pl.ds(k * 16, 16)] = jnp.zeros((16,), jnp.float32)
        return 0
    lax.fori_loop(0, _STEPS, zstep, 0)
    zbase = sid * _ZSL
    pltpu.sync_copy(e_v, accum_sh.at[pl.ds(zbase, _B)])
    pltpu.sync_copy(e_v, accum_sh.at[pl.ds(zbase + _B, _B)])
    pltpu.sync_copy(e_v, accum_sh.at[pl.ds(zbase + 2 * _B, _B)])
    pltpu.sync_copy(e_v.at[pl.ds(0, _ZSL - 3 * _B)],
                    accum_sh.at[pl.ds(zbase + 3 * _B, _ZSL - 3 * _B)])
    plsc.subcore_barrier()

    nblk_mine = (_NBLK - wid + _NW - 1) // _NW

    def blk_body(tt, _):
        off = (wid + tt * _NW) * _B
        pltpu.sync_copy(d_hbm.at[pl.ds(off, _B)], d_v)
        pltpu.sync_copy(ii_hbm.at[pl.ds(off, _B)], ii_v)
        pltpu.sync_copy(ij_hbm.at[pl.ds(off, _B)], ij_v)

        def step(k, _):
            s = pl.ds(k * 16, 16)
            ii = ii_v[s]
            ij = ij_v[s]
            d = d_v[s]
            qi = plsc.load_gather(table_v, [ii])
            qj = plsc.load_gather(table_v, [ij])
            e_v[s] = _energy(d, qi, qj)
            return 0
        lax.fori_loop(0, _STEPS, step, 0)
        pltpu.sync_copy(e_v, accum_sh.at[ii_v], add=True)
        return 0
    lax.fori_loop(0, nblk_mine, blk_body, 0)

    plsc.subcore_barrier()

    def wb(base, n):
        pltpu.sync_copy(accum_sh.at[pl.ds(base, n)], e_v.at[pl.ds(0, n)])

        @pl.when(cid == 0)
        def _():
            pltpu.sync_copy(e_v.at[pl.ds(0, n)], out0_hbm.at[pl.ds(base, n)])

        @pl.when(cid == 1)
        def _():
            pltpu.sync_copy(e_v.at[pl.ds(0, n)], out1_hbm.at[pl.ds(base, n)])
    wb(zbase, _B)
    wb(zbase + _B, _B)
    wb(zbase + 2 * _B, _B)
    wb(zbase + 3 * _B, _ZSL - 3 * _B)


@functools.partial(
    pl.kernel,
    out_type=jax.ShapeDtypeStruct((_N_PAD,), jnp.float32),
    mesh=_mesh,
    compiler_params=pltpu.CompilerParams(needs_layout_passes=False),
    scratch_types=[
        pltpu.VMEM((_CSL,), jnp.float32),
        pltpu.VMEM((_CSL,), jnp.float32),
    ],
)
def _sc_combine(p0_hbm, p1_hbm, out_hbm, a_v, b_v):
    cid = lax.axis_index("c")
    sid = lax.axis_index("s")
    wid = sid * _NC + cid
    base = wid * _CSL
    pltpu.sync_copy(p0_hbm.at[pl.ds(base, _CSL)], a_v)
    pltpu.sync_copy(p1_hbm.at[pl.ds(base, _CSL)], b_v)

    def step(k, _):
        s = pl.ds(k * 16, 16)
        a_v[s] = a_v[s] + b_v[s]
        return 0
    lax.fori_loop(0, _CSL // 16, step, 0)
    pltpu.sync_copy(a_v, out_hbm.at[pl.ds(base, _CSL)])


def kernel(atomic_charges, distances, idx_i, idx_j):
    p0, p1 = _sc_main(atomic_charges, distances, idx_i, idx_j)
    summed = _sc_combine(p0, p1)
    return summed[:_N_NODES]


# speedup vs baseline: 881.6271x; 2.6843x over previous
import functools

import jax
import jax.numpy as jnp
from jax import lax
from jax.experimental import pallas as pl
from jax.experimental.pallas import tpu as pltpu
from jax.experimental.pallas import tpu_sc as plsc

_N_NODES = 100000
_N_EDGES = 6400000
_NC = 2
_NS = 16
_NW = _NC * _NS
_B = 2048
_NBLK = _N_EDGES // _B
_STEPS = _B // 16
_NT_MAX = 99
_N_PAD = 100352
_ZSL = _N_PAD // _NS
_CSL = _N_PAD // _NW

_KEHALF = 7.199822675975274
_C1 = 1.0 / 144.0
_C2 = 1.0 / 6.0
_CUTOFF = 12.0

_mesh = plsc.VectorSubcoreMesh(
    core_axis_name="c", subcore_axis_name="s", num_cores=_NC, num_subcores=_NS
)


def _energy(d, qi, qj):
    t = d * d + 1.0
    ib = plsc.bitcast(t, jnp.int32)
    ib = jnp.int32(0x5F3759DF) - lax.shift_right_arithmetic(ib, 1)
    r = plsc.bitcast(ib, jnp.float32)
    th = 0.5 * t
    r = r * (1.5 - th * r * r)
    r = r * (1.5 - th * r * r)
    ds_ = t * r
    e_shd = r + (ds_ * jnp.float32(_C1) - jnp.float32(_C2))
    e_ord = 1.0 / d + (d * jnp.float32(_C1) - jnp.float32(_C2))
    x = jnp.minimum(d * 0.5, 1.0)
    x2 = x * x
    x3 = x2 * x
    w_on = x3 * ((6.0 * x2 - 15.0 * x) + 10.0)
    mix = e_shd + w_on * (e_ord - e_shd)
    e = (jnp.float32(_KEHALF) * qi) * qj * mix
    return jnp.where(d <= jnp.float32(_CUTOFF), e, jnp.float32(0.0))


@functools.partial(
    pl.kernel,
    out_type=(jax.ShapeDtypeStruct((_N_PAD,), jnp.float32),
              jax.ShapeDtypeStruct((_N_PAD,), jnp.float32)),
    mesh=_mesh,
    compiler_params=pltpu.CompilerParams(needs_layout_passes=False),
    scratch_types=[
        pltpu.VMEM((_N_NODES,), jnp.float32),
        pltpu.VMEM((_B,), jnp.float32),
        pltpu.VMEM((_B,), jnp.float32),
        pltpu.VMEM((_B,), jnp.float32),
        pltpu.VMEM((_B,), jnp.int32),
        pltpu.VMEM((_B,), jnp.int32),
        pltpu.VMEM((_B,), jnp.int32),
        pltpu.VMEM((_B,), jnp.int32),
        pltpu.VMEM((_B,), jnp.int32),
        pltpu.VMEM((_B,), jnp.int32),
        pltpu.VMEM((_B,), jnp.float32),
        pltpu.VMEM((_B,), jnp.float32),
        pltpu.VMEM((_B,), jnp.float32),
        pltpu.VMEM_SHARED((_N_PAD,), jnp.float32),
        pltpu.SemaphoreType.DMA,
        pltpu.SemaphoreType.DMA,
        pltpu.SemaphoreType.DMA,
        pltpu.SemaphoreType.DMA,
        pltpu.SemaphoreType.DMA,
        pltpu.SemaphoreType.DMA,
    ],
)
def _sc_main(ch_hbm, d_hbm, ii_hbm, ij_hbm, out0_hbm, out1_hbm,
             table_v, d0, d1, d2, i0, i1, i2, j0, j1, j2, e0, e1, e2,
             accum_sh, si0, si1, si2, ss0, ss1, ss2):
    cid = lax.axis_index("c")
    sid = lax.axis_index("s")
    wid = sid * _NC + cid
    sets = ((d0, i0, j0, e0, si0, ss0),
            (d1, i1, j1, e1, si1, ss1),
            (d2, i2, j2, e2, si2, ss2))

    pltpu.sync_copy(ch_hbm, table_v)

    def zstep(k, _):
        e0[pl.ds(k * 16, 16)] = jnp.zeros((16,), jnp.float32)
        return 0
    lax.fori_loop(0, _STEPS, zstep, 0)
    zbase = sid * _ZSL
    pltpu.sync_copy(e0, accum_sh.at[pl.ds(zbase, _B)])
    pltpu.sync_copy(e0, accum_sh.at[pl.ds(zbase + _B, _B)])
    pltpu.sync_copy(e0, accum_sh.at[pl.ds(zbase + 2 * _B, _B)])
    pltpu.sync_copy(e0.at[pl.ds(0, _ZSL - 3 * _B)],
                    accum_sh.at[pl.ds(zbase + 3 * _B, _ZSL - 3 * _B)])
    plsc.subcore_barrier()

    nt = (_NBLK - wid + _NW - 1) // _NW

    def issue_load(t, st):
        dv, iv, jv, _, sin, _ = st
        off = (wid + t * _NW) * _B
        pltpu.async_copy(d_hbm.at[pl.ds(off, _B)], dv, sin)
        pltpu.async_copy(ii_hbm.at[pl.ds(off, _B)], iv, sin)
        pltpu.async_copy(ij_hbm.at[pl.ds(off, _B)], jv, sin)

    def wait_load(st):
        dv, iv, jv, _, sin, _ = st
        pltpu.make_async_copy(d_hbm.at[pl.ds(0, _B)], dv, sin).wait()
        pltpu.make_async_copy(ii_hbm.at[pl.ds(0, _B)], iv, sin).wait()
        pltpu.make_async_copy(ij_hbm.at[pl.ds(0, _B)], jv, sin).wait()

    def wait_scatter(st):
        _, iv, _, ev, _, ssc = st
        pltpu.make_async_copy(ev, accum_sh.at[iv], ssc).wait()

    def substep(t, k):
        st = sets[k]
        nxt = sets[(k + 1) % 3]
        @pl.when(jnp.logical_and(t >= 2, t - 2 < nt))
        def _():
            wait_scatter(nxt)

        @pl.when(t + 1 < nt)
        def _():
            issue_load(t + 1, nxt)

        @pl.when(t < nt)
        def _():
            dv, iv, jv, ev, _, ssc = st
            wait_load(st)

            def step(kk, _):
                base = kk * 32
                for u in range(2):
                    s = pl.ds(base + u * 16, 16)
                    qi = plsc.load_gather(table_v, [iv[s]])
                    qj = plsc.load_gather(table_v, [jv[s]])
                    ev[s] = _energy(dv[s], qi, qj)
                return 0
            lax.fori_loop(0, _STEPS // 2, step, 0)
            pltpu.async_copy(ev, accum_sh.at[iv], ssc, add=True)

    issue_load(0, sets[0])

    def trip(g, _):
        substep(3 * g, 0)
        substep(3 * g + 1, 1)
        substep(3 * g + 2, 2)
        return 0
    lax.fori_loop(0, _NT_MAX // 3, trip, 0)

    @pl.when(nt == 98)
    def _():
        wait_scatter(sets[97 % 3])

    plsc.subcore_barrier()

    def wb(base, n):
        pltpu.sync_copy(accum_sh.at[pl.ds(base, n)], e0.at[pl.ds(0, n)])

        @pl.when(cid == 0)
        def _():
            pltpu.sync_copy(e0.at[pl.ds(0, n)], out0_hbm.at[pl.ds(base, n)])

        @pl.when(cid == 1)
        def _():
            pltpu.sync_copy(e0.at[pl.ds(0, n)], out1_hbm.at[pl.ds(base, n)])
    wb(zbase, _B)
    wb(zbase + _B, _B)
    wb(zbase + 2 * _B, _B)
    wb(zbase + 3 * _B, _ZSL - 3 * _B)


@functools.partial(
    pl.kernel,
    out_type=jax.ShapeDtypeStruct((_N_PAD,), jnp.float32),
    mesh=_mesh,
    compiler_params=pltpu.CompilerParams(needs_layout_passes=False),
    scratch_types=[
        pltpu.VMEM((_CSL,), jnp.float32),
        pltpu.VMEM((_CSL,), jnp.float32),
    ],
)
def _sc_combine(p0_hbm, p1_hbm, out_hbm, a_v, b_v):
    cid = lax.axis_index("c")
    sid = lax.axis_index("s")
    wid = sid * _NC + cid
    base = wid * _CSL
    pltpu.sync_copy(p0_hbm.at[pl.ds(base, _CSL)], a_v)
    pltpu.sync_copy(p1_hbm.at[pl.ds(base, _CSL)], b_v)

    def step(k, _):
        s = pl.ds(k * 16, 16)
        a_v[s] = a_v[s] + b_v[s]
        return 0
    lax.fori_loop(0, _CSL // 16, step, 0)
    pltpu.sync_copy(a_v, out_hbm.at[pl.ds(base, _CSL)])


def kernel(atomic_charges, distances, idx_i, idx_j):
    p0, p1 = _sc_main(atomic_charges, distances, idx_i, idx_j)
    summed = _sc_combine(p0, p1)
    return summed[:_N_NODES]
